# baseline (speedup 1.0000x reference)
import functools

import jax
import jax.numpy as jnp
from jax.experimental import pallas as pl
from jax.experimental.pallas import tpu as pltpu

_COST_CLASS = 2.0
_COST_BBOX = 5.0
_COST_GIOU = 2.0

_BLOCK_C = 2048


def _cost_kernel(logits_ref, pbox_ref, isz_ref, tcol_ref, out_ref,
                 *, n_classes, block_c):
    x = logits_ref[0]
    m = jnp.max(x, axis=1, keepdims=True)
    e = jnp.exp(x - m)
    s = jnp.sum(e, axis=1, keepdims=True)
    q = e * (-_COST_CLASS / s)

    ids = tcol_ref[8:9, :].astype(jnp.int32)
    cls = jax.lax.broadcasted_iota(jnp.int32, (n_classes, block_c), 0)
    sel = (cls == ids).astype(jnp.float32)
    acc = jnp.dot(q, sel, preferred_element_type=jnp.float32)

    pr = pbox_ref[0]
    inv = 1.0 / isz_ref[0]
    pn = (_COST_BBOX * pr) * inv
    tc = tcol_ref[...]

    for c in range(4):
        b_n = (_COST_BBOX * tc[c:c + 1, :]) * tc[c + 4:c + 5, :]
        acc = acc + jnp.abs(pn[:, c:c + 1] - b_n)

    ax1 = pr[:, 0:1]
    ay1 = pr[:, 1:2]
    ax2 = pr[:, 2:3]
    ay2 = pr[:, 3:4]
    bx1 = tc[0:1, :]
    by1 = tc[1:2, :]
    bx2 = tc[2:3, :]
    by2 = tc[3:4, :]
    area_a = (ax2 - ax1) * (ay2 - ay1)
    area_b = (bx2 - bx1) * (by2 - by1)

    max_x1 = jnp.maximum(ax1, bx1)
    min_x1 = jnp.minimum(ax1, bx1)
    max_x2 = jnp.maximum(ax2, bx2)
    min_x2 = jnp.minimum(ax2, bx2)
    max_y1 = jnp.maximum(ay1, by1)
    min_y1 = jnp.minimum(ay1, by1)
    max_y2 = jnp.maximum(ay2, by2)
    min_y2 = jnp.minimum(ay2, by2)

    iw = jnp.maximum(min_x2 - max_x1, 0.0)
    ih = jnp.maximum(min_y2 - max_y1, 0.0)
    inter = iw * ih
    union = (area_a + area_b) - inter
    area_e = (max_x2 - min_x1) * (max_y2 - min_y1)

    acc = acc + (-_COST_GIOU * inter) / union
    acc = acc + (_COST_GIOU * (area_e - union)) / area_e
    out_ref[0] = acc


def kernel(pred_logits, pred_boxes, tgt_labels, tgt_boxes,
           image_size_xyxy, image_size_xyxy_tgt):
    b, q, nc = pred_logits.shape
    t = tgt_labels.shape[1]
    bt = b * t

    f32 = jnp.float32
    isz = image_size_xyxy.reshape(b, 1, 4).astype(f32)

    inv_tgt = 1.0 / image_size_xyxy_tgt
    tcol8 = jnp.concatenate(
        [tgt_boxes.reshape(bt, 4), inv_tgt.reshape(bt, 4)], axis=-1)
    tcol8 = tcol8.T
    ids_row = tgt_labels.reshape(1, bt).astype(f32)
    tcol = jnp.concatenate(
        [tcol8, ids_row, jnp.zeros((7, bt), f32)], axis=0)

    grid = (b, bt // _BLOCK_C)
    out = pl.pallas_call(
        functools.partial(_cost_kernel, n_classes=nc, block_c=_BLOCK_C),
        out_shape=jax.ShapeDtypeStruct((b, q, bt), f32),
        grid=grid,
        in_specs=[
            pl.BlockSpec((1, q, nc), lambda i, j: (i, 0, 0)),
            pl.BlockSpec((1, q, 4), lambda i, j: (i, 0, 0)),
            pl.BlockSpec((1, 1, 4), lambda i, j: (i, 0, 0)),
            pl.BlockSpec((16, _BLOCK_C), lambda i, j: (0, j)),
        ],
        out_specs=pl.BlockSpec((1, q, _BLOCK_C), lambda i, j: (i, 0, j)),
        compiler_params=pltpu.CompilerParams(
            dimension_semantics=("parallel", "arbitrary"),
            vmem_limit_bytes=60 * 1024 * 1024,
        ),
        name="hungarian_cost",
    )(pred_logits.astype(f32), pred_boxes.astype(f32), isz, tcol)

    return out


# speedup vs baseline: 1.0396x; 1.0396x over previous
import functools

import jax
import jax.numpy as jnp
from jax.experimental import pallas as pl
from jax.experimental.pallas import tpu as pltpu

_COST_CLASS = 2.0
_COST_BBOX = 5.0
_COST_GIOU = 2.0

_BLOCK_C = 512
_BGRP = 8


def _cost_kernel(logits_ref, pbox_ref, isz_ref, tcol_ref, out_ref,
                 *, n_classes, block_c):
    ids = tcol_ref[8:9, :].astype(jnp.int32)
    cls = jax.lax.broadcasted_iota(jnp.int32, (n_classes, block_c), 0)
    sel = (cls == ids).astype(jnp.float32)

    tc = tcol_ref[...]
    bx1 = tc[0:1, :]
    by1 = tc[1:2, :]
    bx2 = tc[2:3, :]
    by2 = tc[3:4, :]
    area_b = (bx2 - bx1) * (by2 - by1)
    tn = [(_COST_BBOX * tc[c:c + 1, :]) * tc[c + 4:c + 5, :] for c in range(4)]

    for bi in range(_BGRP):
        x = logits_ref[bi]
        m = jnp.max(x, axis=1, keepdims=True)
        e = jnp.exp(x - m)
        s = jnp.sum(e, axis=1, keepdims=True)
        q = e * (-_COST_CLASS / s)
        acc = jnp.dot(q, sel, preferred_element_type=jnp.float32)

        pr = pbox_ref[bi]
        inv = 1.0 / isz_ref[bi]
        pn = (_COST_BBOX * pr) * inv

        for c in range(4):
            acc = acc + jnp.abs(pn[:, c:c + 1] - tn[c])

        ax1 = pr[:, 0:1]
        ay1 = pr[:, 1:2]
        ax2 = pr[:, 2:3]
        ay2 = pr[:, 3:4]
        area_a = (ax2 - ax1) * (ay2 - ay1)

        iw = jnp.maximum(
            jnp.minimum(ax2, bx2) - jnp.maximum(ax1, bx1), 0.0)
        ih = jnp.maximum(
            jnp.minimum(ay2, by2) - jnp.maximum(ay1, by1), 0.0)
        inter = iw * ih
        union = (area_a + area_b) - inter
        area_e = ((jnp.maximum(ax2, bx2) - jnp.minimum(ax1, bx1)) *
                  (jnp.maximum(ay2, by2) - jnp.minimum(ay1, by1)))

        acc = acc + (-_COST_GIOU * inter) / union
        acc = acc + (_COST_GIOU * (area_e - union)) / area_e
        out_ref[:, bi, :] = acc


def kernel(pred_logits, pred_boxes, tgt_labels, tgt_boxes,
           image_size_xyxy, image_size_xyxy_tgt):
    b, q, nc = pred_logits.shape
    t = tgt_labels.shape[1]
    bt = b * t

    f32 = jnp.float32
    isz = image_size_xyxy.reshape(b, 1, 4).astype(f32)

    inv_tgt = 1.0 / image_size_xyxy_tgt
    tcol8 = jnp.concatenate(
        [tgt_boxes.reshape(bt, 4), inv_tgt.reshape(bt, 4)], axis=-1)
    tcol8 = tcol8.T
    ids_row = tgt_labels.reshape(1, bt).astype(f32)
    tcol = jnp.concatenate(
        [tcol8, ids_row, jnp.zeros((7, bt), f32)], axis=0)

    grid = (b // _BGRP, bt // _BLOCK_C)
    out = pl.pallas_call(
        functools.partial(_cost_kernel, n_classes=nc, block_c=_BLOCK_C),
        out_shape=jax.ShapeDtypeStruct((q, b, bt), f32),
        grid=grid,
        in_specs=[
            pl.BlockSpec((_BGRP, q, nc), lambda i, j: (i, 0, 0)),
            pl.BlockSpec((_BGRP, q, 4), lambda i, j: (i, 0, 0)),
            pl.BlockSpec((_BGRP, 1, 4), lambda i, j: (i, 0, 0)),
            pl.BlockSpec((16, _BLOCK_C), lambda i, j: (0, j)),
        ],
        out_specs=pl.BlockSpec(
            (q, _BGRP, _BLOCK_C), lambda i, j: (0, i, j)),
        compiler_params=pltpu.CompilerParams(
            dimension_semantics=("parallel", "arbitrary"),
            vmem_limit_bytes=60 * 1024 * 1024,
        ),
        name="hungarian_cost",
    )(pred_logits.astype(f32), pred_boxes.astype(f32), isz, tcol)

    return out.transpose(1, 0, 2)
